# baseline (speedup 1.0000x reference)
import functools

import jax
import jax.numpy as jnp
from jax import lax
from jax.experimental import pallas as pl
from jax.experimental.pallas import tpu as pltpu
from jax.experimental.pallas import tpu_sc as plsc

N = 10000
E = 320000
C_IN = 128
C = 64
D_E = 16
L = 3
G = 64
POST = 64

NC = 2
NS = 16
NW = NC * NS
EPW = E // NW
CH = 80
NPAD = 10240
RPS = NPAD // NS


def _bn(h, g, b):
    m = jnp.mean(h, axis=0, keepdims=True)
    v = jnp.mean((h - m) ** 2, axis=0, keepdims=True)
    return (h - m) / jnp.sqrt(v + 1e-5) * g + b


def _bdot(a, b):
    return jnp.dot(a.astype(jnp.bfloat16), b.astype(jnp.bfloat16),
                   preferred_element_type=jnp.float32)


def _pre_body(x_ref, w_ref, b_ref, g_ref, bb_ref, wd_ref, ws_ref,
              h_out, pd_out, ps_out):
    h = _bdot(x_ref[...], w_ref[...])
    h = h + b_ref[...]
    h = jax.nn.relu(_bn(h, g_ref[...], bb_ref[...]))
    h_out[...] = h
    pd_out[...] = _bdot(h, wd_ref[...])
    ps_out[...] = _bdot(h, ws_ref[...])


def _pre_call(x, pre_W, pre_b, pre_g, pre_bb, Wd, Ws):
    return pl.pallas_call(
        _pre_body,
        out_shape=(
            jax.ShapeDtypeStruct((N, C), jnp.float32),
            jax.ShapeDtypeStruct((N, 2 * C), jnp.float32),
            jax.ShapeDtypeStruct((N, 2 * C), jnp.float32),
        ),
    )(x, pre_W, pre_b, pre_g, pre_bb, Wd, Ws)


RB = 4000


def _msg_body(a_ref, ea_ref, we_ref, bc_ref, out_ref):
    z = a_ref[...] + _bdot(ea_ref[...], we_ref[...]) + bc_ref[...]
    zf = z[:, :C]
    zs = z[:, C:]
    sig = 1.0 / (1.0 + jnp.exp(-zf))
    sp = jnp.maximum(zs, 0.0) + jnp.log1p(jnp.exp(-jnp.abs(zs)))
    out_ref[:, :C] = sig * sp
    out_ref[:, C:] = jnp.zeros((RB, C), jnp.float32)


def _msg_call(A, ea, We, bcat):
    ne = A.shape[0]
    return pl.pallas_call(
        _msg_body,
        grid=(ne // RB,),
        in_specs=[
            pl.BlockSpec((RB, 2 * C), lambda i: (i, 0)),
            pl.BlockSpec((RB, D_E), lambda i: (i, 0)),
            pl.BlockSpec((D_E, 2 * C), lambda i: (0, 0)),
            pl.BlockSpec((1, 2 * C), lambda i: (0, 0)),
        ],
        out_specs=pl.BlockSpec((RB, 2 * C), lambda i: (i, 0)),
        out_shape=jax.ShapeDtypeStruct((ne, 2 * C), jnp.float32),
    )(A, ea, We, bcat)


def _comb_body(h_ref, pa_ref, pb_ref, g_ref, b_ref, wd_ref, ws_ref,
               h_out, pd_out, ps_out):
    h = (h_ref[...] + pa_ref[0, :N, :C] + pa_ref[1, :N, :C]
         + pb_ref[0, :N, :C] + pb_ref[1, :N, :C])
    h = jax.nn.relu(_bn(h, g_ref[...], b_ref[...]))
    h_out[...] = h
    pd_out[...] = _bdot(h, wd_ref[...])
    ps_out[...] = _bdot(h, ws_ref[...])


def _comb_call(H, pa, pb, g, b, Wd, Ws):
    return pl.pallas_call(
        _comb_body,
        out_shape=(
            jax.ShapeDtypeStruct((N, C), jnp.float32),
            jax.ShapeDtypeStruct((N, 2 * C), jnp.float32),
            jax.ShapeDtypeStruct((N, 2 * C), jnp.float32),
        ),
    )(H, pa, pb, g, b, Wd, Ws)


def _final_body(h_ref, pa_ref, pb2_ref, g_ref, b_ref, batch_ref, pw_ref,
                pb_ref, pg_ref, pbb_ref, ow_ref, ob_ref, out_ref):
    h = (h_ref[...] + pa_ref[0, :N, :C] + pa_ref[1, :N, :C]
         + pb2_ref[0, :N, :C] + pb2_ref[1, :N, :C])
    h = jax.nn.relu(_bn(h, g_ref[...], b_ref[...]))
    gids = lax.broadcasted_iota(jnp.int32, (N, G), 1)
    oh = (batch_ref[...] == gids).astype(jnp.float32)
    sums = lax.dot_general(oh, h, (((0,), (0,)), ((), ())),
                           preferred_element_type=jnp.float32, precision=lax.Precision.HIGHEST)
    cnt = lax.dot_general(oh, jnp.ones((N, 1), jnp.float32),
                          (((0,), (0,)), ((), ())),
                          preferred_element_type=jnp.float32, precision=lax.Precision.HIGHEST)
    pooled = sums / jnp.maximum(cnt, 1.0)
    h1 = _bdot(pooled, pw_ref[...])
    h1 = h1 + pb_ref[...]
    h1 = jax.nn.relu(_bn(h1, pg_ref[...], pbb_ref[...]))
    y = _bdot(h1, ow_ref[...])
    out_ref[...] = y + ob_ref[...]


def _final_call(H, pa, pb, g, b, batch2d, post_W, post_b, post_g, post_bb,
                out_W, out_b):
    return pl.pallas_call(
        _final_body,
        out_shape=jax.ShapeDtypeStruct((G, 1), jnp.float32),
    )(H, pa, pb, g, b, batch2d, post_W, post_b, post_g, post_bb, out_W, out_b)


NCH = 125


def _make_gather_body(ch, epw):
    def _gather_body(pd_hbm, ps_hbm, dst3_hbm, src3_hbm, out_hbm,
                     idxd2, idxs2, bufd0, bufd1, bufs0, bufs1, obuf0, obuf1,
                     semg0, semg1, semw0, semw1):
        c = lax.axis_index("c")
        s = lax.axis_index("s")
        w = s * NC + c
        base0 = w * epw
        bufd = (bufd0, bufd1)
        bufs = (bufs0, bufs1)
        obuf = (obuf0, obuf1)
        semg = (semg0, semg1)
        semw = (semw0, semw1)
        pltpu.sync_copy(dst3_hbm.at[w], idxd2)
        pltpu.sync_copy(src3_hbm.at[w], idxs2)

        def issue(i, b):
            pltpu.async_copy(pd_hbm.at[idxd2.at[i]], bufd[b], semg[b])
            pltpu.async_copy(ps_hbm.at[idxs2.at[i]], bufs[b], semg[b])

        def slot(i, b):
            pltpu.make_async_copy(pd_hbm.at[idxd2.at[i]], bufd[b],
                                  semg[b]).wait()
            pltpu.make_async_copy(ps_hbm.at[idxs2.at[i]], bufs[b],
                                  semg[b]).wait()

            @pl.when(i >= 2)
            def _():
                pltpu.make_async_copy(
                    obuf[b], out_hbm.at[pl.ds(base0, ch)], semw[b]).wait()

            def add_row(r, carry2):
                for cc in range(2 * C // 16):
                    sl = pl.ds(cc * 16, 16)
                    obuf[b][r, sl] = bufd[b][r, sl] + bufs[b][r, sl]
                return carry2

            lax.fori_loop(0, ch, add_row, 0)
            pltpu.async_copy(obuf[b], out_hbm.at[pl.ds(base0 + i * ch, ch)],
                             semw[b])

            @pl.when(i + 2 < NCH)
            def _():
                issue(i + 2, b)

        issue(0, 0)
        issue(1, 1)

        def pair(j2, carry):
            slot(2 * j2, 0)
            slot(2 * j2 + 1, 1)
            return carry

        lax.fori_loop(0, NCH // 2, pair, 0)
        slot(NCH - 1, 0)
        pltpu.make_async_copy(obuf[1], out_hbm.at[pl.ds(base0, ch)],
                              semw[1]).wait()
        pltpu.make_async_copy(obuf[0], out_hbm.at[pl.ds(base0, ch)],
                              semw[0]).wait()

    return _gather_body


def _make_scatter_body(ch, epw):
    def _scatter_body(msg_hbm, dst3_hbm, zeros_hbm, out_hbm, idx2,
                      mb0, mb1, mb2, semm0, semm1, semm2,
                      semsc0, semsc1, semsc2, acc):
        c = lax.axis_index("c")
        s = lax.axis_index("s")
        w = s * NC + c
        base0 = w * epw
        mbuf = (mb0, mb1, mb2)
        semm = (semm0, semm1, semm2)
        semsc = (semsc0, semsc1, semsc2)
        stripe = pl.ds(s * RPS, RPS)
        pltpu.sync_copy(dst3_hbm.at[w], idx2)
        pltpu.sync_copy(zeros_hbm.at[stripe], acc.at[stripe])
        plsc.subcore_barrier()

        def load(i, b):
            pltpu.async_copy(msg_hbm.at[pl.ds(base0 + i * ch, ch)], mbuf[b],
                             semm[b])

        def slot(i, b):
            bn1 = (b + 1) % 3

            @pl.when(i >= 2)
            def _():
                pltpu.make_async_copy(mbuf[bn1], acc.at[idx2.at[0]],
                                      semsc[bn1]).wait()

            @pl.when(i + 1 < NCH)
            def _():
                load(i + 1, bn1)

            pltpu.make_async_copy(msg_hbm.at[pl.ds(base0 + i * ch, ch)],
                                  mbuf[b], semm[b]).wait()
            pltpu.async_copy(mbuf[b], acc.at[idx2.at[i]], semsc[b], add=True)

        load(0, 0)

        def triple(j3, carry):
            for b in range(3):
                slot(3 * j3 + b, b)
            return carry

        lax.fori_loop(0, NCH // 3, triple, 0)
        slot(NCH - 2, (NCH - 2) % 3)
        slot(NCH - 1, (NCH - 1) % 3)
        pltpu.make_async_copy(mbuf[(NCH - 2) % 3], acc.at[idx2.at[0]],
                              semsc[(NCH - 2) % 3]).wait()
        pltpu.make_async_copy(mbuf[(NCH - 1) % 3], acc.at[idx2.at[0]],
                              semsc[(NCH - 1) % 3]).wait()
        plsc.subcore_barrier()
        pltpu.sync_copy(acc.at[stripe], out_hbm.at[c, stripe])

    return _scatter_body


@functools.cache
def _sc_kernels(ch):
    epw = NCH * ch
    ne = epw * NW
    mesh = plsc.VectorSubcoreMesh(core_axis_name="c", subcore_axis_name="s")
    gather = pl.kernel(
        _make_gather_body(ch, epw),
        out_type=jax.ShapeDtypeStruct((ne, 2 * C), jnp.float32),
        mesh=mesh,
        scratch_types=(
            [pltpu.VMEM((NCH, ch), jnp.int32)] * 2
            + [pltpu.VMEM((ch, 2 * C), jnp.float32)] * 6
            + [pltpu.SemaphoreType.DMA] * 4
        ),
    )
    scatter = pl.kernel(
        _make_scatter_body(ch, epw),
        out_type=jax.ShapeDtypeStruct((NC, NPAD, 2 * C), jnp.float32),
        mesh=mesh,
        scratch_types=(
            [pltpu.VMEM((NCH, ch), jnp.int32)]
            + [pltpu.VMEM((ch, 2 * C), jnp.float32)] * 3
            + [pltpu.SemaphoreType.DMA] * 6
            + [pltpu.VMEM_SHARED((NPAD, 2 * C), jnp.float32)]
        ),
    )
    return gather, scatter


EH = E // 2
CHH = 40


def kernel(x, edge_attr, pre_W, pre_b, pre_g, pre_bb, conv_Wf, conv_bf,
           conv_Ws, conv_bs, bn_g, bn_b, post_W, post_b, post_g, post_bb,
           out_W, out_b, edge_index, batch):
    src = edge_index[0]
    dst = edge_index[1]
    d3 = [dst[h * EH:(h + 1) * EH].reshape(NW, NCH, CHH) for h in range(2)]
    s3 = [src[h * EH:(h + 1) * EH].reshape(NW, NCH, CHH) for h in range(2)]
    ea = [edge_attr[h * EH:(h + 1) * EH] for h in range(2)]
    Wd = [jnp.concatenate([conv_Wf[l, :C], conv_Ws[l, :C]], axis=1)
          for l in range(L)]
    Wsrc = [jnp.concatenate([conv_Wf[l, C:2 * C], conv_Ws[l, C:2 * C]], axis=1)
            for l in range(L)]
    We = [jnp.concatenate([conv_Wf[l, 2 * C:], conv_Ws[l, 2 * C:]], axis=1)
          for l in range(L)]
    bcat = [jnp.concatenate([conv_bf[l], conv_bs[l]])[None, :]
            for l in range(L)]
    zeros_nc = jnp.zeros((NPAD, 2 * C), jnp.float32)
    batch2d = batch.astype(jnp.int32).reshape(N, 1)

    gather_kernel, scatter_kernel = _sc_kernels(CHH)
    H, Pd, Ps = _pre_call(x, pre_W, pre_b[None, :], pre_g[None, :],
                          pre_bb[None, :], Wd[0], Wsrc[0])
    for l in range(L):
        A = [gather_kernel(Pd, Ps, d3[h], s3[h]) for h in range(2)]
        msg = [_msg_call(A[h], ea[h], We[l], bcat[l]) for h in range(2)]
        p = [scatter_kernel(msg[h], d3[h], zeros_nc) for h in range(2)]
        if l + 1 < L:
            H, Pd, Ps = _comb_call(H, p[0], p[1], bn_g[l][None, :],
                                   bn_b[l][None, :], Wd[l + 1], Wsrc[l + 1])
        else:
            y = _final_call(H, p[0], p[1], bn_g[l][None, :], bn_b[l][None, :],
                            batch2d, post_W, post_b[None, :],
                            post_g[None, :], post_bb[None, :],
                            out_W, out_b[None, :])
    return y


# speedup vs baseline: 1.0440x; 1.0440x over previous
import functools

import jax
import jax.numpy as jnp
from jax import lax
from jax.experimental import pallas as pl
from jax.experimental.pallas import tpu as pltpu
from jax.experimental.pallas import tpu_sc as plsc

N = 10000
E = 320000
C_IN = 128
C = 64
D_E = 16
L = 3
G = 64
POST = 64

NC = 2
NS = 16
NW = NC * NS
EPW = E // NW
CH = 80
NPAD = 10240
RPS = NPAD // NS


def _bn(h, g, b):
    m = jnp.mean(h, axis=0, keepdims=True)
    v = jnp.mean((h - m) ** 2, axis=0, keepdims=True)
    return (h - m) / jnp.sqrt(v + 1e-5) * g + b


def _bdot(a, b):
    return jnp.dot(a.astype(jnp.bfloat16), b.astype(jnp.bfloat16),
                   preferred_element_type=jnp.float32)


def _pre_body(x_ref, w_ref, b_ref, g_ref, bb_ref, wd_ref, ws_ref,
              h_out, pd_out, ps_out):
    h = _bdot(x_ref[...], w_ref[...])
    h = h + b_ref[...]
    h = jax.nn.relu(_bn(h, g_ref[...], bb_ref[...]))
    h_out[...] = h
    pd_out[...] = _bdot(h, wd_ref[...])
    ps_out[...] = _bdot(h, ws_ref[...])


def _pre_call(x, pre_W, pre_b, pre_g, pre_bb, Wd, Ws):
    return pl.pallas_call(
        _pre_body,
        out_shape=(
            jax.ShapeDtypeStruct((N, C), jnp.float32),
            jax.ShapeDtypeStruct((N, 2 * C), jnp.float32),
            jax.ShapeDtypeStruct((N, 2 * C), jnp.float32),
        ),
    )(x, pre_W, pre_b, pre_g, pre_bb, Wd, Ws)


RB = 2560


def _msg_body(a_ref, ea_ref, we_ref, bc_ref, out_ref):
    z = a_ref[...] + _bdot(ea_ref[...], we_ref[...]) + bc_ref[...]
    zf = z[:, :C]
    zs = z[:, C:]
    sig = 1.0 / (1.0 + jnp.exp(-zf))
    sp = jnp.maximum(zs, 0.0) + jnp.log1p(jnp.exp(-jnp.abs(zs)))
    out_ref[:, :C] = sig * sp
    out_ref[:, C:] = jnp.zeros((RB, C), jnp.float32)


def _msg_call(A, ea, We, bcat):
    ne = A.shape[0]
    return pl.pallas_call(
        _msg_body,
        grid=(ne // RB,),
        in_specs=[
            pl.BlockSpec((RB, 2 * C), lambda i: (i, 0)),
            pl.BlockSpec((RB, D_E), lambda i: (i, 0)),
            pl.BlockSpec((D_E, 2 * C), lambda i: (0, 0)),
            pl.BlockSpec((1, 2 * C), lambda i: (0, 0)),
        ],
        out_specs=pl.BlockSpec((RB, 2 * C), lambda i: (i, 0)),
        out_shape=jax.ShapeDtypeStruct((ne, 2 * C), jnp.float32),
    )(A, ea, We, bcat)


def _comb_body(h_ref, pa_ref, pb_ref, g_ref, b_ref, wd_ref, ws_ref,
               h_out, pd_out, ps_out):
    h = (h_ref[...] + pa_ref[0, :N, :C] + pa_ref[1, :N, :C]
         + pb_ref[0, :N, :C] + pb_ref[1, :N, :C])
    h = jax.nn.relu(_bn(h, g_ref[...], b_ref[...]))
    h_out[...] = h
    pd_out[...] = _bdot(h, wd_ref[...])
    ps_out[...] = _bdot(h, ws_ref[...])


def _comb_call(H, pa, pb, g, b, Wd, Ws):
    return pl.pallas_call(
        _comb_body,
        out_shape=(
            jax.ShapeDtypeStruct((N, C), jnp.float32),
            jax.ShapeDtypeStruct((N, 2 * C), jnp.float32),
            jax.ShapeDtypeStruct((N, 2 * C), jnp.float32),
        ),
    )(H, pa, pb, g, b, Wd, Ws)


def _final_body(h_ref, pa_ref, pb2_ref, g_ref, b_ref, batch_ref, pw_ref,
                pb_ref, pg_ref, pbb_ref, ow_ref, ob_ref, out_ref):
    h = (h_ref[...] + pa_ref[0, :N, :C] + pa_ref[1, :N, :C]
         + pb2_ref[0, :N, :C] + pb2_ref[1, :N, :C])
    h = jax.nn.relu(_bn(h, g_ref[...], b_ref[...]))
    gids = lax.broadcasted_iota(jnp.int32, (N, G), 1)
    oh = (batch_ref[...] == gids).astype(jnp.float32)
    sums = lax.dot_general(oh, h, (((0,), (0,)), ((), ())),
                           preferred_element_type=jnp.float32, precision=lax.Precision.HIGHEST)
    cnt = lax.dot_general(oh, jnp.ones((N, 1), jnp.float32),
                          (((0,), (0,)), ((), ())),
                          preferred_element_type=jnp.float32, precision=lax.Precision.HIGHEST)
    pooled = sums / jnp.maximum(cnt, 1.0)
    h1 = _bdot(pooled, pw_ref[...])
    h1 = h1 + pb_ref[...]
    h1 = jax.nn.relu(_bn(h1, pg_ref[...], pbb_ref[...]))
    y = _bdot(h1, ow_ref[...])
    out_ref[...] = y + ob_ref[...]


def _final_call(H, pa, pb, g, b, batch2d, post_W, post_b, post_g, post_bb,
                out_W, out_b):
    return pl.pallas_call(
        _final_body,
        out_shape=jax.ShapeDtypeStruct((G, 1), jnp.float32),
    )(H, pa, pb, g, b, batch2d, post_W, post_b, post_g, post_bb, out_W, out_b)


def _make_gather_body(ch, epw, nch):
    def _gather_body(pd_hbm, ps_hbm, dst3_hbm, src3_hbm, out_hbm,
                     idxd2, idxs2, bufd0, bufd1, bufs0, bufs1, obuf0, obuf1,
                     semg0, semg1, semw0, semw1):
        c = lax.axis_index("c")
        s = lax.axis_index("s")
        w = s * NC + c
        base0 = w * epw
        bufd = (bufd0, bufd1)
        bufs = (bufs0, bufs1)
        obuf = (obuf0, obuf1)
        semg = (semg0, semg1)
        semw = (semw0, semw1)
        pltpu.sync_copy(dst3_hbm.at[w], idxd2)
        pltpu.sync_copy(src3_hbm.at[w], idxs2)

        def issue(i, b):
            pltpu.async_copy(pd_hbm.at[idxd2.at[i]], bufd[b], semg[b])
            pltpu.async_copy(ps_hbm.at[idxs2.at[i]], bufs[b], semg[b])

        def slot(i, b):
            pltpu.make_async_copy(pd_hbm.at[idxd2.at[i]], bufd[b],
                                  semg[b]).wait()
            pltpu.make_async_copy(ps_hbm.at[idxs2.at[i]], bufs[b],
                                  semg[b]).wait()

            @pl.when(i >= 2)
            def _():
                pltpu.make_async_copy(
                    obuf[b], out_hbm.at[pl.ds(base0, ch)], semw[b]).wait()

            def add_row(r, carry2):
                for cc in range(2 * C // 16):
                    sl = pl.ds(cc * 16, 16)
                    obuf[b][r, sl] = bufd[b][r, sl] + bufs[b][r, sl]
                return carry2

            lax.fori_loop(0, ch, add_row, 0)
            pltpu.async_copy(obuf[b], out_hbm.at[pl.ds(base0 + i * ch, ch)],
                             semw[b])

            @pl.when(i + 2 < nch)
            def _():
                issue(i + 2, b)

        issue(0, 0)
        issue(1, 1)

        def pair(j2, carry):
            slot(2 * j2, 0)
            slot(2 * j2 + 1, 1)
            return carry

        lax.fori_loop(0, nch // 2, pair, 0)
        for i in range(2 * (nch // 2), nch):
            slot(i, i % 2)
        pltpu.make_async_copy(obuf[1], out_hbm.at[pl.ds(base0, ch)],
                              semw[1]).wait()
        pltpu.make_async_copy(obuf[0], out_hbm.at[pl.ds(base0, ch)],
                              semw[0]).wait()

    return _gather_body


def _make_scatter_body(ch, epw, nch):
    def _scatter_body(msg_hbm, dst3_hbm, zeros_hbm, out_hbm, idx2,
                      mb0, mb1, mb2, semm0, semm1, semm2,
                      semsc0, semsc1, semsc2, acc):
        c = lax.axis_index("c")
        s = lax.axis_index("s")
        w = s * NC + c
        base0 = w * epw
        mbuf = (mb0, mb1, mb2)
        semm = (semm0, semm1, semm2)
        semsc = (semsc0, semsc1, semsc2)
        stripe = pl.ds(s * RPS, RPS)
        pltpu.sync_copy(dst3_hbm.at[w], idx2)
        pltpu.sync_copy(zeros_hbm.at[stripe], acc.at[stripe])
        plsc.subcore_barrier()

        def load(i, b):
            pltpu.async_copy(msg_hbm.at[pl.ds(base0 + i * ch, ch)], mbuf[b],
                             semm[b])

        def slot(i, b):
            bn1 = (b + 1) % 3

            @pl.when(i >= 2)
            def _():
                pltpu.make_async_copy(mbuf[bn1], acc.at[idx2.at[0]],
                                      semsc[bn1]).wait()

            @pl.when(i + 1 < nch)
            def _():
                load(i + 1, bn1)

            pltpu.make_async_copy(msg_hbm.at[pl.ds(base0 + i * ch, ch)],
                                  mbuf[b], semm[b]).wait()
            pltpu.async_copy(mbuf[b], acc.at[idx2.at[i]], semsc[b], add=True)

        load(0, 0)

        def triple(j3, carry):
            for b in range(3):
                slot(3 * j3 + b, b)
            return carry

        lax.fori_loop(0, nch // 3, triple, 0)
        for i in range(3 * (nch // 3), nch):
            slot(i, i % 3)
        pltpu.make_async_copy(mbuf[(nch - 2) % 3], acc.at[idx2.at[0]],
                              semsc[(nch - 2) % 3]).wait()
        pltpu.make_async_copy(mbuf[(nch - 1) % 3], acc.at[idx2.at[0]],
                              semsc[(nch - 1) % 3]).wait()
        plsc.subcore_barrier()
        pltpu.sync_copy(acc.at[stripe], out_hbm.at[c, stripe])

    return _scatter_body


@functools.cache
def _sc_kernels(ch, nch):
    epw = nch * ch
    ne = epw * NW
    mesh = plsc.VectorSubcoreMesh(core_axis_name="c", subcore_axis_name="s")
    gather = pl.kernel(
        _make_gather_body(ch, epw, nch),
        out_type=jax.ShapeDtypeStruct((ne, 2 * C), jnp.float32),
        mesh=mesh,
        scratch_types=(
            [pltpu.VMEM((nch, ch), jnp.int32)] * 2
            + [pltpu.VMEM((ch, 2 * C), jnp.float32)] * 6
            + [pltpu.SemaphoreType.DMA] * 4
        ),
    )
    scatter = pl.kernel(
        _make_scatter_body(ch, epw, nch),
        out_type=jax.ShapeDtypeStruct((NC, NPAD, 2 * C), jnp.float32),
        mesh=mesh,
        scratch_types=(
            [pltpu.VMEM((nch, ch), jnp.int32)]
            + [pltpu.VMEM((ch, 2 * C), jnp.float32)] * 3
            + [pltpu.SemaphoreType.DMA] * 6
            + [pltpu.VMEM_SHARED((NPAD, 2 * C), jnp.float32)]
        ),
    )
    return gather, scatter


HNCH = (64, 61)
HE = (NW * 64 * CH, NW * 61 * CH)
HOFF = (0, HE[0])


def kernel(x, edge_attr, pre_W, pre_b, pre_g, pre_bb, conv_Wf, conv_bf,
           conv_Ws, conv_bs, bn_g, bn_b, post_W, post_b, post_g, post_bb,
           out_W, out_b, edge_index, batch):
    src = edge_index[0]
    dst = edge_index[1]
    d3 = [dst[HOFF[h]:HOFF[h] + HE[h]].reshape(NW, HNCH[h], CH)
          for h in range(2)]
    s3 = [src[HOFF[h]:HOFF[h] + HE[h]].reshape(NW, HNCH[h], CH)
          for h in range(2)]
    ea = [edge_attr[HOFF[h]:HOFF[h] + HE[h]] for h in range(2)]
    Wd = [jnp.concatenate([conv_Wf[l, :C], conv_Ws[l, :C]], axis=1)
          for l in range(L)]
    Wsrc = [jnp.concatenate([conv_Wf[l, C:2 * C], conv_Ws[l, C:2 * C]], axis=1)
            for l in range(L)]
    We = [jnp.concatenate([conv_Wf[l, 2 * C:], conv_Ws[l, 2 * C:]], axis=1)
          for l in range(L)]
    bcat = [jnp.concatenate([conv_bf[l], conv_bs[l]])[None, :]
            for l in range(L)]
    zeros_nc = jnp.zeros((NPAD, 2 * C), jnp.float32)
    batch2d = batch.astype(jnp.int32).reshape(N, 1)

    sck = [_sc_kernels(CH, HNCH[h]) for h in range(2)]
    H, Pd, Ps = _pre_call(x, pre_W, pre_b[None, :], pre_g[None, :],
                          pre_bb[None, :], Wd[0], Wsrc[0])
    for l in range(L):
        A = [sck[h][0](Pd, Ps, d3[h], s3[h]) for h in range(2)]
        msg = [_msg_call(A[h], ea[h], We[l], bcat[l]) for h in range(2)]
        p = [sck[h][1](msg[h], d3[h], zeros_nc) for h in range(2)]
        if l + 1 < L:
            H, Pd, Ps = _comb_call(H, p[0], p[1], bn_g[l][None, :],
                                   bn_b[l][None, :], Wd[l + 1], Wsrc[l + 1])
        else:
            y = _final_call(H, p[0], p[1], bn_g[l][None, :], bn_b[l][None, :],
                            batch2d, post_W, post_b[None, :],
                            post_g[None, :], post_bb[None, :],
                            out_W, out_b[None, :])
    return y
